# baseline (speedup 1.0000x reference)
import functools

import jax
import jax.numpy as jnp
from jax import lax
from jax.experimental import pallas as pl
from jax.experimental.pallas import tpu as pltpu
from jax.experimental.pallas import tpu_sc as plsc

_NV = 1000
_NF = 26
_CD = 32
_B = 1024
_NW = 32
_RPW = _B // _NW
_ROWSTRIDE = _NF * (_NV + 1)
_NROW = (_NV + 1) * _NF * _NF
_NPACK = _NROW // 4


_CHU = 1024
_NCH = _NROW // _CHU
_TAILM = _NROW - _NCH * _CHU
_NPACKP = (_NCH * _CHU + 896) // 4


def _repack_body(wt3, wlin, in_v, out_v, sem):
    wid = lax.axis_index("s") * 2 + lax.axis_index("c")
    lanes = lax.iota(jnp.int32, 16)

    def _emit(n_m):
        @pl.loop(0, n_m // 16)
        def _(o):
            ml = o * 16 + lanes
            pr = lax.shift_right_logical(ml, 2)
            plane = lax.shift_left(jnp.bitwise_and(ml, 3), 5)
            for cg in range(4):
                for ci in range(8):
                    val = in_v[cg, ci, pl.ds(o * 16, 16)]
                    plsc.store_scatter(out_v, [pr, plane + (cg * 8 + ci)],
                                       val)

    @pl.loop(wid, _NCH, step=_NW)
    def _(i):
        m0 = pl.multiple_of(i * _CHU, 128)
        for cg in range(4):
            pltpu.sync_copy(wt3.at[cg, :, pl.ds(m0, _CHU)], in_v.at[cg])
        _emit(_CHU)
        pltpu.sync_copy(out_v, wlin.at[pl.ds(i * (_CHU // 4), _CHU // 4)])

    @pl.when(wid == 0)
    def _():
        m0 = pl.multiple_of(_NCH * _CHU, 128)
        for cg in range(4):
            pltpu.sync_copy(wt3.at[cg, :, pl.ds(m0, 896)],
                            in_v.at[cg, :, pl.ds(0, 896)])
        _emit(896)
        pltpu.sync_copy(out_v.at[pl.ds(0, 224)],
                        wlin.at[pl.ds(_NCH * (_CHU // 4), 224)])


_repack_cache = []


def _repack(wt3):
    if not _repack_cache:
        _repack_cache.append(functools.partial(
            pl.kernel,
            out_type=jax.ShapeDtypeStruct((_NPACKP, 128), jnp.float32),
            mesh=plsc.VectorSubcoreMesh(core_axis_name="c",
                                        subcore_axis_name="s"),
            compiler_params=pltpu.CompilerParams(needs_layout_passes=False,
                                                 use_tc_tiling_on_sc=True),
            scratch_types=[
                pltpu.VMEM((4, 8, _CHU), jnp.float32),
                pltpu.VMEM((_CHU // 4, 128), jnp.float32),
                pltpu.SemaphoreType.DMA,
            ],
        )(_repack_body))
    return _repack_cache[0](wt3)


def _sc_body(x_t, i_t, s_t, wlin, b_hbm, wm_hbm, bv_hbm,
             xT_v, iT_v, sT_v, bias_v, idx_v, off_v, ipos_v, rows_v,
             wm_v, bv_v, sem):
    wid = lax.axis_index("s") * 2 + lax.axis_index("c")
    base = wid * _RPW
    slab = pl.multiple_of((wid // 4) * 128, 128)
    q = (wid % 4) * _RPW

    pltpu.sync_copy(x_t.at[:, pl.ds(slab, 128)], xT_v)
    pltpu.sync_copy(i_t.at[:, pl.ds(slab, 128)], iT_v)
    pltpu.sync_copy(s_t.at[:, pl.ds(slab, 128)], sT_v)
    pltpu.sync_copy(b_hbm, bias_v)

    lanes = lax.iota(jnp.int32, 16)
    for g in range(_RPW // 16):
        sl = pl.ds(q + g * 16, 16)
        best_v = iT_v[0, sl]
        best_i = jnp.zeros((16,), jnp.int32)
        for k in range(1, _NF):
            v = iT_v[k, sl]
            m = v > best_v
            best_i = jnp.where(m, jnp.full((16,), k, jnp.int32), best_i)
            best_v = jnp.where(m, v, best_v)
        ipos_v[pl.ds(g * 16, 16)] = best_i
        ibase = best_i * _ROWSTRIDE
        rows16 = jnp.full((16,), g * 16, jnp.int32) + lanes
        for k in range(_NF):
            xv = xT_v[k, sl]
            sv = sT_v[k, sl]
            col = xv * sv + (1 - sv) * _NV
            ridx = ibase + k * (_NV + 1) + col
            idx_v[k, pl.ds(g * 16, 16)] = lax.shift_right_logical(ridx, 2)
            off_v[k, pl.ds(g * 16, 16)] = (
                lax.shift_left(jnp.bitwise_and(ridx, 3), 5))
        for c in range(_CD):
            bvv = plsc.load_gather(bias_v,
                                   [best_i, jnp.full((16,), c, jnp.int32)])
            plsc.store_scatter(bv_v, [rows16, jnp.full((16,), c, jnp.int32)],
                               bvv)

    inv = jnp.float32(1.0 / _NF)
    for g in range(_RPW // 16):
        copies = [pltpu.async_copy(wlin.at[idx_v.at[k, pl.ds(g * 16, 16)]],
                                   rows_v.at[k], sem)
                  for k in range(_NF)]
        for cp in copies:
            cp.wait()

        gsl = pl.ds(g * 16, 16)
        grows = jnp.full((16,), g * 16, jnp.int32) + lanes

        @pl.loop(0, _CD)
        def _(c, _g=g, _sl=gsl, _rows=grows):
            acc = jnp.zeros((16,), jnp.float32)
            for k in range(_NF):
                kv = jnp.full((16,), k, jnp.int32)
                acc = acc + plsc.load_gather(
                    rows_v, [kv, lanes, off_v[k, _sl] + c])
            plsc.store_scatter(wm_v, [_rows, jnp.zeros((16,), jnp.int32) + c],
                               acc * inv)

    pltpu.sync_copy(wm_v, wm_hbm.at[pl.ds(base, _RPW)])
    pltpu.sync_copy(bv_v, bv_hbm.at[pl.ds(base, _RPW)])


_sc_call_cache = []


def _sc_call(*argv):
    if not _sc_call_cache:
        _sc_call_cache.append(functools.partial(
            pl.kernel,
            out_type=(
                jax.ShapeDtypeStruct((_B, _CD), jnp.float32),
                jax.ShapeDtypeStruct((_B, _CD), jnp.float32),
            ),
            mesh=plsc.VectorSubcoreMesh(core_axis_name="c",
                                        subcore_axis_name="s"),
            compiler_params=pltpu.CompilerParams(needs_layout_passes=False,
                                                 use_tc_tiling_on_sc=True),
            scratch_types=[
                pltpu.VMEM((_NF, 128), jnp.int32),
                pltpu.VMEM((_NF, 128), jnp.float32),
                pltpu.VMEM((_NF, 128), jnp.int32),
                pltpu.VMEM((_NF, _CD), jnp.float32),
                pltpu.VMEM((_NF, _RPW), jnp.int32),
                pltpu.VMEM((_NF, _RPW), jnp.int32),
                pltpu.VMEM((_RPW,), jnp.int32),
                pltpu.VMEM((_NF, 16, 128), jnp.float32),
                pltpu.VMEM((_RPW, _CD), jnp.float32),
                pltpu.VMEM((_RPW, _CD), jnp.float32),
                pltpu.SemaphoreType.DMA,
            ],
        )(_sc_body))
    return _sc_call_cache[0](*argv)


def _tc_body(wm_ref, bv_ref, out_ref):
    out_ref[...] = wm_ref[...][None, :, :] + bv_ref[...][:, :, None]


_BI = 16


def _tc_call(wmT, bv):
    return pl.pallas_call(
        _tc_body,
        grid=(_B // _BI,),
        in_specs=[
            pl.BlockSpec((_CD, _B), lambda i: (0, 0)),
            pl.BlockSpec((_BI, _CD), lambda i: (i, 0)),
        ],
        out_specs=pl.BlockSpec((_BI, _CD, _B), lambda i: (i, 0, 0)),
        out_shape=jax.ShapeDtypeStruct((_B, _CD, _B), jnp.float32),
    )(wmT, bv)


@jax.jit
def kernel(X, I, S, weights, bias):
    X = X.astype(jnp.int32)
    S = S.astype(jnp.int32)
    wt3 = weights.T.reshape(4, 8, _NROW)
    wlin = _repack(wt3)
    wm, bv = _sc_call(X.T, I.T, S.T, wlin, bias)
    out_t = _tc_call(wm.T, bv)
    return out_t.transpose(0, 2, 1)


# speedup vs baseline: 1.5900x; 1.5900x over previous
import functools

import jax
import jax.numpy as jnp
from jax import lax
from jax.experimental import pallas as pl
from jax.experimental.pallas import tpu as pltpu
from jax.experimental.pallas import tpu_sc as plsc

_NV = 1000
_NF = 26
_CD = 32
_B = 1024
_NW = 32
_RPW = _B // _NW
_ROWSTRIDE = _NF * (_NV + 1)


def _sc_body(x_t, i_t, s_t, w_hbm, b_hbm, wm_hbm, bv_hbm,
             xT_v, iT_v, sT_v, idx_v, ipos_v, rows_v, wm_v, bv_v, sem):
    wid = lax.axis_index("s") * 2 + lax.axis_index("c")
    base = wid * _RPW

    pltpu.sync_copy(x_t.at[:, pl.ds(base, _RPW)], xT_v)
    pltpu.sync_copy(i_t.at[:, pl.ds(base, _RPW)], iT_v)
    pltpu.sync_copy(s_t.at[:, pl.ds(base, _RPW)], sT_v)

    for g in range(_RPW // 16):
        sl = pl.ds(g * 16, 16)
        best_v = iT_v[0, sl]
        best_i = jnp.zeros((16,), jnp.int32)
        for k in range(1, _NF):
            v = iT_v[k, sl]
            m = v > best_v
            best_i = jnp.where(m, jnp.full((16,), k, jnp.int32), best_i)
            best_v = jnp.where(m, v, best_v)
        ipos_v[pl.ds(g * 16, 16)] = best_i
        ibase = best_i * _ROWSTRIDE
        for k in range(_NF):
            xv = xT_v[k, sl]
            sv = sT_v[k, sl]
            col = xv * sv + (1 - sv) * _NV
            idx_v[k, pl.ds(g * 16, 16)] = ibase + k * (_NV + 1) + col

    copies = [pltpu.async_copy(w_hbm.at[idx_v.at[k]], rows_v.at[k], sem)
              for k in range(_NF)]
    bcopy = pltpu.async_copy(b_hbm.at[ipos_v], bv_v, sem)
    for cp in copies:
        cp.wait()
    bcopy.wait()

    inv = jnp.float32(1.0 / _NF)

    @pl.loop(0, _RPW)
    def _(r):
        acc0 = jnp.zeros((16,), jnp.float32)
        acc1 = jnp.zeros((16,), jnp.float32)
        for k in range(_NF):
            acc0 = acc0 + rows_v[k, r, pl.ds(0, 16)]
            acc1 = acc1 + rows_v[k, r, pl.ds(16, 16)]
        wm_v[r, pl.ds(0, 16)] = acc0 * inv
        wm_v[r, pl.ds(16, 16)] = acc1 * inv

    pltpu.sync_copy(wm_v, wm_hbm.at[pl.ds(base, _RPW)])
    pltpu.sync_copy(bv_v, bv_hbm.at[pl.ds(base, _RPW)])


_sc_call_cache = []


def _sc_call(*argv):
    if not _sc_call_cache:
        _sc_call_cache.append(functools.partial(
            pl.kernel,
            out_type=(
                jax.ShapeDtypeStruct((_B, _CD), jnp.float32),
                jax.ShapeDtypeStruct((_B, _CD), jnp.float32),
            ),
            mesh=plsc.VectorSubcoreMesh(core_axis_name="c",
                                        subcore_axis_name="s"),
            compiler_params=pltpu.CompilerParams(needs_layout_passes=False,
                                                 use_tc_tiling_on_sc=False),
            scratch_types=[
                pltpu.VMEM((_NF, _RPW), jnp.int32),
                pltpu.VMEM((_NF, _RPW), jnp.float32),
                pltpu.VMEM((_NF, _RPW), jnp.int32),
                pltpu.VMEM((_NF, _RPW), jnp.int32),
                pltpu.VMEM((_RPW,), jnp.int32),
                pltpu.VMEM((_NF, _RPW, _CD), jnp.float32),
                pltpu.VMEM((_RPW, _CD), jnp.float32),
                pltpu.VMEM((_RPW, _CD), jnp.float32),
                pltpu.SemaphoreType.DMA,
            ],
        )(_sc_body))
    return _sc_call_cache[0](*argv)


def _tc_body(wm_ref, bv_ref, out_ref):
    out_ref[...] = wm_ref[...][None, :, :] + bv_ref[...][:, :, None]


_BI = 16


def _tc_call(wmT, bv):
    return pl.pallas_call(
        _tc_body,
        grid=(_B // _BI,),
        in_specs=[
            pl.BlockSpec((_CD, _B), lambda i: (0, 0)),
            pl.BlockSpec((_BI, _CD), lambda i: (i, 0)),
        ],
        out_specs=pl.BlockSpec((_BI, _CD, _B), lambda i: (i, 0, 0)),
        out_shape=jax.ShapeDtypeStruct((_B, _CD, _B), jnp.float32),
    )(wmT, bv)


@jax.jit
def kernel(X, I, S, weights, bias):
    X = X.astype(jnp.int32)
    S = S.astype(jnp.int32)
    wm, bv = _sc_call(X.T, I.T, S.T, weights, bias)
    out_t = _tc_call(wm.T, bv)
    return out_t.transpose(0, 2, 1)
